# baseline (speedup 1.0000x reference)
import functools

import jax
import jax.numpy as jnp
from jax import lax
from jax.experimental import pallas as pl
from jax.experimental.pallas import tpu as pltpu
from jax.experimental.pallas import tpu_sc as plsc

BATCH = 16384
DIM = 32
NROWS = 1000000
NC = 2
NS = 16
NW = NC * NS
BPW = BATCH // NW
CHUNK = 128
NCHUNK = BPW // CHUNK

BLK = 32768
SLAB = 1024
NSLAB = 12
CPT = SLAB * NSLAB
C_SC = CPT * NW
TC_OFF = C_SC // BLK
NBLK_TC = (NROWS - C_SC + BLK - 1) // BLK
C_MASK = (1 << 18) - 1


def _matvec_body(tm_ref, tu_ref, wm_ref, wu_ref, sm_ref, su_ref):
  sm_ref[...] = jnp.dot(wm_ref[...], tm_ref[...],
                        preferred_element_type=jnp.float32)[0]
  su_ref[...] = jnp.dot(wu_ref[...], tu_ref[...],
                        preferred_element_type=jnp.float32)[0]


def _matvec_tc(tm, tu, wm, wu):
  return pl.pallas_call(
      _matvec_body,
      grid=(NBLK_TC,),
      in_specs=[
          pl.BlockSpec((DIM, BLK), lambda i: (0, i + TC_OFF)),
          pl.BlockSpec((DIM, BLK), lambda i: (0, i + TC_OFF)),
          pl.BlockSpec((8, DIM), lambda i: (0, 0)),
          pl.BlockSpec((8, DIM), lambda i: (0, 0)),
      ],
      out_specs=[
          pl.BlockSpec((BLK,), lambda i: (i + TC_OFF,)),
          pl.BlockSpec((BLK,), lambda i: (i + TC_OFF,)),
      ],
      out_shape=[
          jax.ShapeDtypeStruct((NROWS,), jnp.float32),
          jax.ShapeDtypeStruct((NROWS,), jnp.float32),
      ],
  )(tm, tu, wm, wu)


def _sc_matvec_body(tm_hbm, tu_hbm, wb_hbm, sm_hbm, su_hbm,
                    wb_v, buf_a, buf_b, out_v, sem_a, sem_b):
  wid = lax.axis_index("s") * NC + lax.axis_index("c")
  colbase = pl.multiple_of(wid * CPT, CPT)
  pltpu.sync_copy(wb_hbm, wb_v)

  bufs = (buf_a, buf_b)
  sems = (sem_a, sem_b)

  def fire(tab, par, c0):
    c0 = pl.multiple_of(c0, SLAB)
    for dt in range(4):
      pltpu.async_copy(tab.at[pl.ds(dt * 8, 8), pl.ds(c0, SLAB)],
                       bufs[par].at[dt], sems[par])

  def wait_buf(tab, par):
    for dt in range(4):
      pltpu.make_async_copy(tab.at[pl.ds(dt * 8, 8), pl.ds(colbase, SLAB)],
                            bufs[par].at[dt], sems[par]).wait()

  def compute_slab(par, so, ws):
    b = bufs[par]
    nq = 8

    def grp(g, carry):
      o = pl.multiple_of(g * (16 * nq), 16)
      accs = [jnp.zeros((16,), jnp.float32)] * nq
      for d in range(DIM):
        w = ws[d]
        for q in range(nq):
          accs[q] = accs[q] + b[d // 8, d % 8, pl.ds(o + q * 16, 16)] * w
      for q in range(nq):
        out_v[pl.ds(so + o + q * 16, 16)] = accs[q]
      return carry

    lax.fori_loop(0, SLAB // (16 * nq), grp, 0)

  def do_table(tab, wofs, out_hbm):
    ws = tuple(wb_v[wofs + d] for d in range(DIM))
    fire(tab, 0, colbase)

    def pair(i, carry):
      c0a = colbase + (2 * i) * SLAB
      fire(tab, 1, c0a + SLAB)
      wait_buf(tab, 0)
      compute_slab(0, (2 * i) * SLAB, ws)
      fire(tab, 0, c0a + 2 * SLAB)
      wait_buf(tab, 1)
      compute_slab(1, (2 * i + 1) * SLAB, ws)
      return carry

    lax.fori_loop(0, NSLAB // 2, pair, 0)
    wait_buf(tab, 0)
    pltpu.sync_copy(out_v, out_hbm.at[pl.ds(colbase, CPT)])

  do_table(tm_hbm, 0, sm_hbm)
  do_table(tu_hbm, DIM, su_hbm)


def _matvec_sc(tm, tu, wb):
  mesh = plsc.VectorSubcoreMesh(core_axis_name="c", subcore_axis_name="s")
  return pl.kernel(
      _sc_matvec_body,
      out_type=[
          jax.ShapeDtypeStruct((C_SC,), jnp.float32),
          jax.ShapeDtypeStruct((C_SC,), jnp.float32),
      ],
      mesh=mesh,
      scratch_types=[
          pltpu.VMEM((2 * DIM, 16), jnp.float32),
          pltpu.VMEM((4, 8, SLAB), jnp.float32),
          pltpu.VMEM((4, 8, SLAB), jnp.float32),
          pltpu.VMEM((CPT,), jnp.float32),
          pltpu.SemaphoreType.DMA,
          pltpu.SemaphoreType.DMA,
      ],
  )(tm, tu, wb)


def _sc_body(midx_hbm, uidx_hbm, smt_hbm, sut_hbm, sms_hbm, sus_hbm, bb_hbm,
             out_hbm, midx_v, uidx_v, midxc_v, uidxc_v,
             gmt_v, gut_v, gms_v, gus_v, bb_v, out_v, sem):
  wid = lax.axis_index("s") * NC + lax.axis_index("c")
  base = pl.multiple_of(wid * BPW, BPW)

  pltpu.sync_copy(midx_hbm.at[pl.ds(base, BPW)], midx_v)
  pltpu.sync_copy(uidx_hbm.at[pl.ds(base, BPW)], uidx_v)
  pltpu.sync_copy(bb_hbm, bb_v)

  for k in range(BPW // 16):
    o = k * 16
    mi = midx_v[pl.ds(o, 16)]
    ui = uidx_v[pl.ds(o, 16)]
    midxc_v[pl.ds(o, 16)] = jnp.where(mi < C_SC, mi, jnp.bitwise_and(mi, C_MASK))
    uidxc_v[pl.ds(o, 16)] = jnp.where(ui < C_SC, ui, jnp.bitwise_and(ui, C_MASK))

  handles = []
  for j in range(NCHUNK):
    o = j * CHUNK
    sl = pl.ds(o, CHUNK)
    handles.append(pltpu.async_copy(smt_hbm.at[midx_v.at[sl]], gmt_v.at[sl], sem))
    handles.append(pltpu.async_copy(sut_hbm.at[uidx_v.at[sl]], gut_v.at[sl], sem))
    handles.append(pltpu.async_copy(sms_hbm.at[midxc_v.at[sl]], gms_v.at[sl], sem))
    handles.append(pltpu.async_copy(sus_hbm.at[uidxc_v.at[sl]], gus_v.at[sl], sem))
  for h in handles:
    h.wait()

  bvec = bb_v[...]
  for k in range(BPW // 16):
    sl = pl.ds(k * 16, 16)
    vm = jnp.where(midx_v[sl] < C_SC, gms_v[sl], gmt_v[sl])
    vu = jnp.where(uidx_v[sl] < C_SC, gus_v[sl], gut_v[sl])
    out_v[sl] = vm + vu + bvec

  pltpu.sync_copy(out_v, out_hbm.at[pl.ds(base, BPW)])


def _sc_gather_add(midx, uidx, sm_tc, su_tc, sm_sc, su_sc, bb):
  mesh = plsc.VectorSubcoreMesh(core_axis_name="c", subcore_axis_name="s")
  return pl.kernel(
      _sc_body,
      out_type=jax.ShapeDtypeStruct((BATCH,), jnp.float32),
      mesh=mesh,
      scratch_types=[
          pltpu.VMEM((BPW,), jnp.int32),
          pltpu.VMEM((BPW,), jnp.int32),
          pltpu.VMEM((BPW,), jnp.int32),
          pltpu.VMEM((BPW,), jnp.int32),
          pltpu.VMEM((BPW,), jnp.float32),
          pltpu.VMEM((BPW,), jnp.float32),
          pltpu.VMEM((BPW,), jnp.float32),
          pltpu.VMEM((BPW,), jnp.float32),
          pltpu.VMEM((16,), jnp.float32),
          pltpu.VMEM((BPW,), jnp.float32),
          pltpu.SemaphoreType.DMA,
      ],
  )(midx, uidx, sm_tc, su_tc, sm_sc, su_sc, bb)


def kernel(user_index, movie_index, movie_table, user_table, W, b):
  tm = movie_table.T
  tu = user_table.T
  wm = jnp.zeros((8, DIM), jnp.float32).at[0].set(W[:DIM, 0])
  wu = jnp.zeros((8, DIM), jnp.float32).at[0].set(W[DIM:, 0])
  wb = jnp.broadcast_to(W.reshape(2 * DIM, 1), (2 * DIM, 16))
  bb = jnp.broadcast_to(b.reshape(1), (16,)).astype(jnp.float32)
  sm_sc, su_sc = _matvec_sc(tm, tu, wb)
  sm_tc, su_tc = _matvec_tc(tm, tu, wm, wu)
  return _sc_gather_add(movie_index.astype(jnp.int32),
                        user_index.astype(jnp.int32),
                        sm_tc, su_tc, sm_sc, su_sc, bb)


# speedup vs baseline: 1.0989x; 1.0989x over previous
import functools

import jax
import jax.numpy as jnp
from jax import lax
from jax.experimental import pallas as pl
from jax.experimental.pallas import tpu as pltpu
from jax.experimental.pallas import tpu_sc as plsc

BATCH = 16384
DIM = 32
NROWS = 1000000
NC = 2
NS = 16
NW = NC * NS
BPW = BATCH // NW
CHUNK = 128
NCHUNK = BPW // CHUNK

BLK = 32768
NBLK = (NROWS + BLK - 1) // BLK


def _matvec_body(tm_ref, tu_ref, wm_ref, wu_ref, sm_ref, su_ref):
  sm_ref[...] = jnp.dot(wm_ref[...], tm_ref[...],
                        preferred_element_type=jnp.float32)[0]
  su_ref[...] = jnp.dot(wu_ref[...], tu_ref[...],
                        preferred_element_type=jnp.float32)[0]


def _matvec(tm, tu, wm, wu):
  return pl.pallas_call(
      _matvec_body,
      grid=(NBLK,),
      in_specs=[
          pl.BlockSpec((DIM, BLK), lambda i: (0, i)),
          pl.BlockSpec((DIM, BLK), lambda i: (0, i)),
          pl.BlockSpec((8, DIM), lambda i: (0, 0)),
          pl.BlockSpec((8, DIM), lambda i: (0, 0)),
      ],
      out_specs=[
          pl.BlockSpec((BLK,), lambda i: (i,)),
          pl.BlockSpec((BLK,), lambda i: (i,)),
      ],
      out_shape=[
          jax.ShapeDtypeStruct((NROWS,), jnp.float32),
          jax.ShapeDtypeStruct((NROWS,), jnp.float32),
      ],
  )(tm, tu, wm, wu)


def _sc_body(midx_hbm, uidx_hbm, sm_hbm, su_hbm, bb_hbm, out_hbm,
             midx_v, uidx_v, sm_v, su_v, bb_v, out_v, sem):
  wid = lax.axis_index("s") * NC + lax.axis_index("c")
  base = pl.multiple_of(wid * BPW, BPW)

  pltpu.sync_copy(midx_hbm.at[pl.ds(base, BPW)], midx_v)
  pltpu.sync_copy(uidx_hbm.at[pl.ds(base, BPW)], uidx_v)
  pltpu.sync_copy(bb_hbm, bb_v)

  handles = []
  for j in range(NCHUNK):
    o = j * CHUNK
    handles.append(pltpu.async_copy(
        sm_hbm.at[midx_v.at[pl.ds(o, CHUNK)]], sm_v.at[pl.ds(o, CHUNK)], sem))
    handles.append(pltpu.async_copy(
        su_hbm.at[uidx_v.at[pl.ds(o, CHUNK)]], su_v.at[pl.ds(o, CHUNK)], sem))
  for h in handles:
    h.wait()

  bvec = bb_v[...]
  for k in range(BPW // 16):
    o = k * 16
    out_v[pl.ds(o, 16)] = sm_v[pl.ds(o, 16)] + su_v[pl.ds(o, 16)] + bvec

  pltpu.sync_copy(out_v, out_hbm.at[pl.ds(base, BPW)])


def _sc_gather_add(midx, uidx, sm, su, bb):
  mesh = plsc.VectorSubcoreMesh(core_axis_name="c", subcore_axis_name="s")
  return pl.kernel(
      _sc_body,
      out_type=jax.ShapeDtypeStruct((BATCH,), jnp.float32),
      mesh=mesh,
      scratch_types=[
          pltpu.VMEM((BPW,), jnp.int32),
          pltpu.VMEM((BPW,), jnp.int32),
          pltpu.VMEM((BPW,), jnp.float32),
          pltpu.VMEM((BPW,), jnp.float32),
          pltpu.VMEM((16,), jnp.float32),
          pltpu.VMEM((BPW,), jnp.float32),
          pltpu.SemaphoreType.DMA,
      ],
  )(midx, uidx, sm, su, bb)


def kernel(user_index, movie_index, movie_table, user_table, W, b):
  tm = movie_table.T
  tu = user_table.T
  wm = jnp.zeros((8, DIM), jnp.float32).at[0].set(W[:DIM, 0])
  wu = jnp.zeros((8, DIM), jnp.float32).at[0].set(W[DIM:, 0])
  bb = jnp.broadcast_to(b.reshape(1), (16,)).astype(jnp.float32)
  sm, su = _matvec(tm, tu, wm, wu)
  return _sc_gather_add(movie_index.astype(jnp.int32),
                        user_index.astype(jnp.int32), sm, su, bb)
